# baseline (speedup 1.0000x reference)
import functools

import jax
import jax.numpy as jnp
from jax import lax
from jax.experimental import pallas as pl
from jax.experimental.pallas import tpu as pltpu
from jax.experimental.pallas import tpu_sc as plsc

_VOCAB = 100000
_BATCH = 1024
_SEQ = 20
_PAD = 1

_NC = 2
_NS = 16
_NW = _NC * _NS
_ROWS_W = _BATCH // _NW
_TOK_W = _ROWS_W * _SEQ
_CHUNK = 128
_NCHUNK = _TOK_W // _CHUNK
_LANES = 16
_ZWORDS = 100000
_SLICE = _ROWS_W * _VOCAB
_NZDMA = _SLICE // _ZWORDS


def _sc_body(ids_hbm, w_hbm, out_hbm, ids_v, w_v, val_v, fidx_v, zero_v,
             gsem, zsem, ssem):
    wid = lax.axis_index("s") * _NC + lax.axis_index("c")
    base_elem = wid * _SLICE
    base_tok = wid * _TOK_W

    pltpu.sync_copy(ids_hbm.at[wid], ids_v)

    gathers = [
        pltpu.async_copy(w_hbm.at[ids_v.at[j]], w_v.at[j], gsem)
        for j in range(_NCHUNK)
    ]

    def _zinit(i, _):
        zero_v[pl.ds(i * _LANES, _LANES)] = jnp.zeros((_LANES,), jnp.float32)
        return 0

    lax.fori_loop(0, _ZWORDS // _LANES, _zinit, 0)
    zeros_out = [
        pltpu.async_copy(
            zero_v, out_hbm.at[pl.ds(base_elem + k * _ZWORDS, _ZWORDS)], zsem)
        for k in range(_NZDMA)
    ]

    for g in gathers:
        g.wait()

    lane = lax.iota(jnp.int32, _LANES)
    for j in range(_NCHUNK):
        for i in range(_CHUNK // _LANES):
            sl = pl.ds(i * _LANES, _LANES)
            idv = ids_v[j, sl]
            wv = w_v[j, sl]
            pos = lane + (base_tok + j * _CHUNK + i * _LANES)
            row = lax.div(pos, _SEQ)
            fidx_v[j, sl] = row * _VOCAB + idv
            val_v[j, sl] = jnp.where(idv == _PAD, jnp.float32(0.0), wv)

    for z in zeros_out:
        z.wait()

    scatters = [
        pltpu.async_copy(val_v.at[j], out_hbm.at[fidx_v.at[j]], ssem)
        for j in range(_NCHUNK)
    ]
    for s in scatters:
        s.wait()


@jax.jit
def _encode(ids_grouped, weights):
    mesh = plsc.VectorSubcoreMesh(
        core_axis_name="c", subcore_axis_name="s")
    run = pl.kernel(
        _sc_body,
        out_type=jax.ShapeDtypeStruct((_BATCH * _VOCAB,), jnp.float32),
        mesh=mesh,
        scratch_types=[
            pltpu.VMEM((_NCHUNK, _CHUNK), jnp.int32),
            pltpu.VMEM((_NCHUNK, _CHUNK), jnp.float32),
            pltpu.VMEM((_NCHUNK, _CHUNK), jnp.float32),
            pltpu.VMEM((_NCHUNK, _CHUNK), jnp.int32),
            pltpu.VMEM((_ZWORDS,), jnp.float32),
            pltpu.SemaphoreType.DMA,
            pltpu.SemaphoreType.DMA,
            pltpu.SemaphoreType.DMA,
        ],
    )
    return run(ids_grouped, weights)


def kernel(input_ids, weights):
    ids_grouped = input_ids.astype(jnp.int32).reshape(_NW, _NCHUNK, _CHUNK)
    out_flat = _encode(ids_grouped, weights)
    return out_flat.reshape(_BATCH, _VOCAB)
